# baseline (speedup 1.0000x reference)
import functools

import jax
import jax.numpy as jnp
from jax import lax
from jax.experimental import pallas as pl
from jax.experimental.pallas import tpu as pltpu
from jax.experimental.pallas import tpu_sc as plsc

F32 = jnp.float32
I32 = jnp.int32

N = 10000
E = 320000
HID = 128
G = 256
EPS = 1e-5

NC, NS = 2, 16
NT = NC * NS
NPAD = 10240
EPAD = 327680
ET = EPAD // NT
CH = ET // 128
CH0, CH1 = 160, 0
EPAD_E = NS * (CH0 + CH1) * 128
NPAD_E = 10112
RPT_E = NPAD_E // NS
RPT = NPAD // NS
NODE_DUMMY = N + 100
GPAD = 12288
NB = 10


def _hist_body(rowi, batv, zrow, ones, degp, cntp,
               deg_sp, cnt_sp, idx_v, bidx_v, rbuf, sem):
    c = lax.axis_index("c")
    s = lax.axis_index("s")
    w = c * NS + s
    pltpu.sync_copy(zrow, rbuf)
    for m in range(RPT // 128):
        pltpu.sync_copy(rbuf, deg_sp.at[pl.ds(s * RPT + m * 128, 128)])

    @pl.when(s < 4)
    def _():
        pltpu.sync_copy(rbuf, cnt_sp.at[pl.ds(s * 128, 128)])

    pltpu.sync_copy(rowi.at[pl.ds(w * CH, CH)], idx_v)

    @pl.when(w < GPAD // 128 // 8)
    def _():
        pltpu.sync_copy(batv.at[pl.ds(w * 8, 8)], bidx_v)

    plsc.subcore_barrier()
    pltpu.sync_copy(ones, rbuf)

    def body(j, carry):
        pltpu.async_copy(rbuf, deg_sp.at[idx_v.at[j]], sem, add=True)
        return carry

    lax.fori_loop(0, CH, body, 0)

    @pl.when(w < GPAD // 128 // 8)
    def _():
        for j in range(8):
            pltpu.async_copy(rbuf, cnt_sp.at[bidx_v.at[j]], sem, add=True)

    def drain(j, carry):
        pltpu.make_async_copy(rbuf, deg_sp.at[idx_v.at[j]], sem).wait()
        return carry

    lax.fori_loop(0, CH, drain, 0)

    @pl.when(w < GPAD // 128 // 8)
    def _():
        for j in range(8):
            pltpu.make_async_copy(rbuf, cnt_sp.at[bidx_v.at[j]], sem).wait()

    plsc.subcore_barrier()
    pltpu.sync_copy(deg_sp.at[pl.ds(s * RPT, RPT)],
                    degp.at[c, pl.ds(s * RPT, RPT)])

    @pl.when(s == 0)
    def _():
        pltpu.sync_copy(cnt_sp, cntp.at[c])


@functools.cache
def _hist():
    mesh = plsc.VectorSubcoreMesh(
        core_axis_name="c", subcore_axis_name="s",
        num_cores=NC, num_subcores=NS)
    return pl.kernel(
        _hist_body,
        out_type=(jax.ShapeDtypeStruct((NC, NPAD, HID), F32),
                  jax.ShapeDtypeStruct((NC, 512, HID), F32)),
        mesh=mesh,
        scratch_types=(
            pltpu.VMEM_SHARED((NPAD, HID), F32),
            pltpu.VMEM_SHARED((512, HID), F32),
            pltpu.VMEM((CH, 128), I32),
            pltpu.VMEM((8, 128), I32),
            pltpu.VMEM((128, HID), F32),
            pltpu.SemaphoreType.DMA,
        ),
    )


def _edge_body(t2d, gidx, colx, zrow, part, agg_sp,
               gi0, gi1, gi2, ci0, ci1, ci2, rb0, rb1, rb2,
               gs0, gs1, gs2, ss0, ss1, ss2):
    c = lax.axis_index("c")
    s = lax.axis_index("s")
    base = s * CH0 * 128
    gis = (gi0, gi1, gi2)
    cis = (ci0, ci1, ci2)
    rbs = (rb0, rb1, rb2)
    gsem = (gs0, gs1, gs2)
    ssem = (ss0, ss1, ss2)
    pltpu.sync_copy(zrow, rb0)
    for m in range(4):
        pltpu.sync_copy(rb0, agg_sp.at[pl.ds(s * RPT_E + m * 128, 128)])
    pltpu.sync_copy(rb0.at[pl.ds(0, RPT_E - 512)],
                    agg_sp.at[pl.ds(s * RPT_E + 512, RPT_E - 512)])
    plsc.subcore_barrier()

    def loadidx(j, k):
        pltpu.sync_copy(gidx.at[pl.ds(base + j * 128, 128)], gis[k])
        pltpu.sync_copy(colx.at[pl.ds(base + j * 128, 128)], cis[k])

    def gather(k):
        pltpu.async_copy(t2d.at[gis[k]], rbs[k], gsem[k])

    @pl.when(c == 0)
    def _():
        loadidx(0, 0)
        gather(0)

        def body(i, carry):
            j0 = i * 2
            loadidx(j0 + 1, 1)
            gather(1)
            pltpu.make_async_copy(t2d.at[gis[0]], rbs[0], gsem[0]).wait()
            pltpu.sync_copy(rbs[0], agg_sp.at[cis[0]], add=True)

            @pl.when(i < CH0 // 2 - 1)
            def _(j0=j0):
                loadidx(j0 + 2, 0)
                gather(0)

            pltpu.make_async_copy(t2d.at[gis[1]], rbs[1], gsem[1]).wait()
            pltpu.sync_copy(rbs[1], agg_sp.at[cis[1]], add=True)
            return carry

        lax.fori_loop(0, CH0 // 2, body, 0)

    plsc.subcore_barrier()
    pltpu.sync_copy(agg_sp.at[pl.ds(s * RPT_E, RPT_E)],
                    part.at[c, pl.ds(s * RPT_E, RPT_E)])


@functools.cache
def _edge():
    mesh = plsc.VectorSubcoreMesh(
        core_axis_name="c", subcore_axis_name="s",
        num_cores=NC, num_subcores=NS)
    return pl.kernel(
        _edge_body,
        out_type=jax.ShapeDtypeStruct((NC, NPAD_E, HID), F32),
        mesh=mesh,
        scratch_types=(
            pltpu.VMEM_SHARED((NPAD_E, HID), F32),
            pltpu.VMEM((128,), I32),
            pltpu.VMEM((128,), I32),
            pltpu.VMEM((128,), I32),
            pltpu.VMEM((128,), I32),
            pltpu.VMEM((128,), I32),
            pltpu.VMEM((128,), I32),
            pltpu.VMEM((128, HID), F32),
            pltpu.VMEM((128, HID), F32),
            pltpu.VMEM((128, HID), F32),
            pltpu.SemaphoreType.DMA,
            pltpu.SemaphoreType.DMA,
            pltpu.SemaphoreType.DMA,
            pltpu.SemaphoreType.DMA,
            pltpu.SemaphoreType.DMA,
            pltpu.SemaphoreType.DMA,
        ),
    )


def _enc_body(xf, dp, a0, a1, a2, a3, a4, a5, a6, a7, a8,
              bd0, bd1, bd2, w1, b1, root1, t_out, self_out, aux_out):
    atoms = (a0, a1, a2, a3, a4, a5, a6, a7, a8)
    xb = xf[...]
    h0 = jnp.zeros((1000, HID), F32)
    for t, tbl in enumerate(atoms):
        base = tbl[0:1, :]
        h0 = h0 + base + xb[:, t:t + 1] * (tbl[1:2, :] - base)
    deg = dp[0, :, 0:1] + dp[1, :, 0:1] + 1.0
    dinv = lax.rsqrt(deg)
    rdeg = 1.0 / deg
    aux_out[...] = jnp.concatenate(
        [dinv, rdeg, jnp.zeros((1000, 6), F32)], axis=1)
    hw = jnp.dot(h0, w1[...], preferred_element_type=F32) + b1[...]
    for ci in range(8):
        e = (bd0[(ci >> 2) & 1:((ci >> 2) & 1) + 1, :]
             + bd1[(ci >> 1) & 1:((ci >> 1) & 1) + 1, :]
             + bd2[ci & 1:(ci & 1) + 1, :])
        t_out[ci] = dinv * jnp.maximum(hw + e, 0.0)
    self_out[...] = jnp.maximum(hw + root1[...], 0.0) * rdeg


def _mid_body(part, self1, aux, h1pre_out, bn_out):
    i = pl.program_id(0)
    p = part[...]
    dinv = aux[:, 0:1]
    h = dinv * (p[0] + p[1]) + self1[...]
    h1pre_out[...] = h
    sm = jnp.sum(h, axis=0, keepdims=True)
    sq = jnp.sum(h * h, axis=0, keepdims=True)

    @pl.when(i == 0)
    def _():
        bn_out[...] = jnp.zeros((8, HID), F32)

    bn_out[0:1, :] = bn_out[0:1, :] + sm
    bn_out[1:2, :] = bn_out[1:2, :] + sq


def _l2_body(h1pre, bn, aux, bd0, bd1, bd2, w2, b2, root2, g1, be1,
             t_out, self_out):
    mean = bn[0:1, :] * (1.0 / N)
    var = bn[1:2, :] * (1.0 / N) - mean * mean
    rstd = lax.rsqrt(var + EPS)
    h1 = jnp.maximum((h1pre[...] - mean) * rstd * g1[...] + be1[...], 0.0)
    dinv = aux[:, 0:1]
    rdeg = aux[:, 1:2]
    hw = jnp.dot(h1, w2[...], preferred_element_type=F32) + b2[...]
    for ci in range(8):
        e = (bd0[(ci >> 2) & 1:((ci >> 2) & 1) + 1, :]
             + bd1[(ci >> 1) & 1:((ci >> 1) & 1) + 1, :]
             + bd2[ci & 1:(ci & 1) + 1, :])
        t_out[ci] = dinv * jnp.maximum(hw + e, 0.0)
    self_out[...] = jnp.maximum(hw + root2[...], 0.0) * rdeg


def _pool_body(part, self2, aux, batr, pool_out, bn_out):
    i = pl.program_id(0)
    p = part[...]
    dinv = aux[:, 0:1]
    h = dinv * (p[0] + p[1]) + self2[...]
    sm = jnp.sum(h, axis=0, keepdims=True)
    sq = jnp.sum(h * h, axis=0, keepdims=True)

    @pl.when(i == 0)
    def _():
        bn_out[...] = jnp.zeros((8, HID), F32)
        pool_out[...] = jnp.zeros((G, HID), F32)

    bn_out[0:1, :] = bn_out[0:1, :] + sm
    bn_out[1:2, :] = bn_out[1:2, :] + sq
    b = batr[0]
    oh = (lax.broadcasted_iota(I32, (G, 1000), 0) == b).astype(F32)
    pool_out[...] = pool_out[...] + jnp.dot(oh, h, preferred_element_type=F32)


def _head_body(pool, bn, cntp, wp, bp, g2, be2, out):
    cnt = cntp[0, 0:G, 0:1] + cntp[1, 0:G, 0:1]
    mean = bn[0:1, :] * (1.0 / N)
    var = bn[1:2, :] * (1.0 / N) - mean * mean
    rstd = lax.rsqrt(var + EPS)
    hgr = pool[...] / jnp.maximum(cnt, 1.0)
    hg = (hgr - mean) * rstd * g2[...] + be2[...]
    hg = jnp.where(cnt > 0.0, hg, 0.0)
    out[...] = jnp.dot(hg, wp[...], preferred_element_type=F32) + bp[...]


def _full(shape):
    nd = len(shape)
    return pl.BlockSpec(shape, lambda i, _nd=nd: (0,) * _nd)


_DP_SPEC = pl.BlockSpec((NC, 1000, HID), lambda i: (0, i, 0))
_ROW_SPEC = pl.BlockSpec((1000, HID), lambda i: (i, 0))
_AUX_SPEC = pl.BlockSpec((1000, 8), lambda i: (i, 0))
_T_SPEC = pl.BlockSpec((8, 1000, HID), lambda i: (0, i, 0))
_ACC8_SPEC = pl.BlockSpec((8, HID), lambda i: (0, 0))


def kernel(params, x, edge_index, edge_attr, batch):
    p = params
    l1, l2 = p['layers'][0], p['layers'][1]

    xf = jnp.pad(x.astype(F32), ((0, 0), (0, 7)))
    row = edge_index[0].astype(I32)
    col = edge_index[1].astype(I32)
    ea = edge_attr.astype(I32)
    code = ea[:, 0] * 4 + ea[:, 1] * 2 + ea[:, 2]
    gidx = code * N + row
    padn = EPAD - E
    padn_e = EPAD_E - E
    gidx1d = jnp.pad(gidx, (0, padn_e))
    col1d = jnp.pad(col, (0, padn_e), constant_values=NODE_DUMMY)
    rowi2d = jnp.pad(row, (0, padn),
                     constant_values=NODE_DUMMY).reshape(EPAD // 128, 128)
    bat2d = jnp.pad(batch.astype(I32), (0, GPAD - N),
                    constant_values=G).reshape(GPAD // 128, 128)
    batr = batch.astype(I32).reshape(NB, 1, 1000)
    zrow = jnp.zeros((128, HID), F32)
    ones = jnp.ones((128, HID), F32)
    r1 = lambda a: a.reshape(1, -1)

    degp, cntp = _hist()(rowi2d, bat2d, zrow, ones)

    t1, self1, aux = pl.pallas_call(
        _enc_body,
        grid=(NB,),
        in_specs=[pl.BlockSpec((1000, 16), lambda i: (i, 0)), _DP_SPEC]
                 + [_full(t.shape) for t in p['atom']]
                 + [_full(t.shape) for t in l1['bond']]
                 + [_full((HID, HID)), _full((1, HID)), _full((1, HID))],
        out_specs=[_T_SPEC, _ROW_SPEC, _AUX_SPEC],
        out_shape=[jax.ShapeDtypeStruct((8, N, HID), F32),
                   jax.ShapeDtypeStruct((N, HID), F32),
                   jax.ShapeDtypeStruct((N, 8), F32)],
    )(xf, degp, *p['atom'], *l1['bond'], l1['W'], r1(l1['b']), l1['root'])

    part1 = _edge()(t1.reshape(8 * N, HID), gidx1d, col1d, zrow)

    h1pre, bn1 = pl.pallas_call(
        _mid_body,
        grid=(NB,),
        in_specs=[_PART_SPEC := pl.BlockSpec((NC, 1000, HID),
                                             lambda i: (0, i, 0)),
                  _ROW_SPEC, _AUX_SPEC],
        out_specs=[_ROW_SPEC, _ACC8_SPEC],
        out_shape=[jax.ShapeDtypeStruct((N, HID), F32),
                   jax.ShapeDtypeStruct((8, HID), F32)],
    )(part1, self1, aux)

    t2, self2 = pl.pallas_call(
        _l2_body,
        grid=(NB,),
        in_specs=[_ROW_SPEC, _ACC8_SPEC, _AUX_SPEC]
                 + [_full(t.shape) for t in l2['bond']]
                 + [_full((HID, HID)), _full((1, HID)), _full((1, HID)),
                    _full((1, HID)), _full((1, HID))],
        out_specs=[_T_SPEC, _ROW_SPEC],
        out_shape=[jax.ShapeDtypeStruct((8, N, HID), F32),
                   jax.ShapeDtypeStruct((N, HID), F32)],
    )(h1pre, bn1, aux, *l2['bond'], l2['W'], r1(l2['b']), l2['root'],
      r1(l1['bn_gamma']), r1(l1['bn_beta']))

    part2 = _edge()(t2.reshape(8 * N, HID), gidx1d, col1d, zrow)

    pool, bn2 = pl.pallas_call(
        _pool_body,
        grid=(NB,),
        in_specs=[_PART_SPEC, _ROW_SPEC, _AUX_SPEC,
                  pl.BlockSpec((1, 1, 1000), lambda i: (i, 0, 0))],
        out_specs=[pl.BlockSpec((G, HID), lambda i: (0, 0)), _ACC8_SPEC],
        out_shape=[jax.ShapeDtypeStruct((G, HID), F32),
                   jax.ShapeDtypeStruct((8, HID), F32)],
    )(part2, self2, aux, batr)

    out = pl.pallas_call(
        _head_body,
        grid=(1,),
        in_specs=[_full((G, HID)), _ACC8_SPEC, _full((NC, 512, HID)),
                  _full((HID, HID)), _full((1, HID)),
                  _full((1, HID)), _full((1, HID))],
        out_specs=pl.BlockSpec((G, HID), lambda i: (0, 0)),
        out_shape=jax.ShapeDtypeStruct((G, HID), F32),
    )(pool, bn2, cntp, p['Wp'], r1(p['bp']),
      r1(l2['bn_gamma']), r1(l2['bn_beta']))
    return out


# speedup vs baseline: 1.4545x; 1.4545x over previous
import functools

import jax
import jax.numpy as jnp
from jax import lax
from jax.experimental import pallas as pl
from jax.experimental.pallas import tpu as pltpu
from jax.experimental.pallas import tpu_sc as plsc

F32 = jnp.float32
I32 = jnp.int32

N = 10000
E = 320000
HID = 128
G = 256
EPS = 1e-5

NC, NS = 2, 16
NT = NC * NS
NPAD = 10240
EPAD = 327680
ET = EPAD // NT
CH = ET // 128
CH0, CH1 = 152, 8
EPAD_E = NS * (CH0 + CH1) * 128
NPAD_E = 10112
RPT_E = NPAD_E // NS
RPT = NPAD // NS
NODE_DUMMY = N + 100
GPAD = 12288
NB = 10


def _hist_body(rowi, batv, zrow, ones, degp, cntp,
               deg_sp, cnt_sp, idx_v, bidx_v, rbuf, sem):
    c = lax.axis_index("c")
    s = lax.axis_index("s")
    w = c * NS + s
    pltpu.sync_copy(zrow, rbuf)
    for m in range(RPT // 128):
        pltpu.sync_copy(rbuf, deg_sp.at[pl.ds(s * RPT + m * 128, 128)])

    @pl.when(s < 4)
    def _():
        pltpu.sync_copy(rbuf, cnt_sp.at[pl.ds(s * 128, 128)])

    pltpu.sync_copy(rowi.at[pl.ds(w * CH, CH)], idx_v)

    @pl.when(w < GPAD // 128 // 8)
    def _():
        pltpu.sync_copy(batv.at[pl.ds(w * 8, 8)], bidx_v)

    plsc.subcore_barrier()
    pltpu.sync_copy(ones, rbuf)

    def body(j, carry):
        pltpu.async_copy(rbuf, deg_sp.at[idx_v.at[j]], sem, add=True)
        return carry

    lax.fori_loop(0, CH, body, 0)

    @pl.when(w < GPAD // 128 // 8)
    def _():
        for j in range(8):
            pltpu.async_copy(rbuf, cnt_sp.at[bidx_v.at[j]], sem, add=True)

    def drain(j, carry):
        pltpu.make_async_copy(rbuf, deg_sp.at[idx_v.at[j]], sem).wait()
        return carry

    lax.fori_loop(0, CH, drain, 0)

    @pl.when(w < GPAD // 128 // 8)
    def _():
        for j in range(8):
            pltpu.make_async_copy(rbuf, cnt_sp.at[bidx_v.at[j]], sem).wait()

    plsc.subcore_barrier()
    pltpu.sync_copy(deg_sp.at[pl.ds(s * RPT, RPT)],
                    degp.at[c, pl.ds(s * RPT, RPT)])

    @pl.when(s == 0)
    def _():
        pltpu.sync_copy(cnt_sp, cntp.at[c])


@functools.cache
def _hist():
    mesh = plsc.VectorSubcoreMesh(
        core_axis_name="c", subcore_axis_name="s",
        num_cores=NC, num_subcores=NS)
    return pl.kernel(
        _hist_body,
        out_type=(jax.ShapeDtypeStruct((NC, NPAD, HID), F32),
                  jax.ShapeDtypeStruct((NC, 512, HID), F32)),
        mesh=mesh,
        scratch_types=(
            pltpu.VMEM_SHARED((NPAD, HID), F32),
            pltpu.VMEM_SHARED((512, HID), F32),
            pltpu.VMEM((CH, 128), I32),
            pltpu.VMEM((8, 128), I32),
            pltpu.VMEM((128, HID), F32),
            pltpu.SemaphoreType.DMA,
        ),
    )


def _edge_body(t2d, gidx, colx, zrow, part, agg_sp,
               gi0, gi1, gi2, ci0, ci1, ci2, rb0, rb1, rb2,
               gs0, gs1, gs2, ss0, ss1, ss2):
    c = lax.axis_index("c")
    s = lax.axis_index("s")
    n_my = jnp.where(c == 0, CH0, CH1)
    base = (jnp.where(c == 0, s * CH0, NS * CH0 + s * CH1)) * 128
    gis = (gi0, gi1, gi2)
    cis = (ci0, ci1, ci2)
    rbs = (rb0, rb1, rb2)
    gsem = (gs0, gs1, gs2)
    ssem = (ss0, ss1, ss2)
    pltpu.sync_copy(zrow, rb0)
    for m in range(4):
        pltpu.sync_copy(rb0, agg_sp.at[pl.ds(s * RPT_E + m * 128, 128)])
    pltpu.sync_copy(rb0.at[pl.ds(0, RPT_E - 512)],
                    agg_sp.at[pl.ds(s * RPT_E + 512, RPT_E - 512)])
    plsc.subcore_barrier()

    def loadidx(j, k):
        pltpu.sync_copy(gidx.at[pl.ds(base + j * 128, 128)], gis[k])
        pltpu.sync_copy(colx.at[pl.ds(base + j * 128, 128)], cis[k])

    def gather(k):
        pltpu.async_copy(t2d.at[gis[k]], rbs[k], gsem[k])

    loadidx(0, 0)
    gather(0)

    def body(i, carry):
        j0 = i * 2
        loadidx(j0 + 1, 1)
        gather(1)
        pltpu.make_async_copy(t2d.at[gis[0]], rbs[0], gsem[0]).wait()
        pltpu.sync_copy(rbs[0], agg_sp.at[cis[0]], add=True)

        @pl.when(i < n_my // 2 - 1)
        def _(j0=j0):
            loadidx(j0 + 2, 0)
            gather(0)

        pltpu.make_async_copy(t2d.at[gis[1]], rbs[1], gsem[1]).wait()
        pltpu.sync_copy(rbs[1], agg_sp.at[cis[1]], add=True)
        return carry

    lax.fori_loop(0, n_my // 2, body, 0)
    plsc.subcore_barrier()
    pltpu.sync_copy(agg_sp.at[pl.ds(s * RPT_E, RPT_E)],
                    part.at[c, pl.ds(s * RPT_E, RPT_E)])


@functools.cache
def _edge():
    mesh = plsc.VectorSubcoreMesh(
        core_axis_name="c", subcore_axis_name="s",
        num_cores=NC, num_subcores=NS)
    return pl.kernel(
        _edge_body,
        out_type=jax.ShapeDtypeStruct((NC, NPAD_E, HID), F32),
        mesh=mesh,
        scratch_types=(
            pltpu.VMEM_SHARED((NPAD_E, HID), F32),
            pltpu.VMEM((128,), I32),
            pltpu.VMEM((128,), I32),
            pltpu.VMEM((128,), I32),
            pltpu.VMEM((128,), I32),
            pltpu.VMEM((128,), I32),
            pltpu.VMEM((128,), I32),
            pltpu.VMEM((128, HID), F32),
            pltpu.VMEM((128, HID), F32),
            pltpu.VMEM((128, HID), F32),
            pltpu.SemaphoreType.DMA,
            pltpu.SemaphoreType.DMA,
            pltpu.SemaphoreType.DMA,
            pltpu.SemaphoreType.DMA,
            pltpu.SemaphoreType.DMA,
            pltpu.SemaphoreType.DMA,
        ),
    )


def _enc_body(xf, dp, a0, a1, a2, a3, a4, a5, a6, a7, a8,
              bd0, bd1, bd2, w1, b1, root1, t_out, self_out, aux_out):
    atoms = (a0, a1, a2, a3, a4, a5, a6, a7, a8)
    xb = xf[...]
    h0 = jnp.zeros((1000, HID), F32)
    for t, tbl in enumerate(atoms):
        base = tbl[0:1, :]
        h0 = h0 + base + xb[:, t:t + 1] * (tbl[1:2, :] - base)
    deg = dp[0, :, 0:1] + dp[1, :, 0:1] + 1.0
    dinv = lax.rsqrt(deg)
    rdeg = 1.0 / deg
    aux_out[...] = jnp.concatenate(
        [dinv, rdeg, jnp.zeros((1000, 6), F32)], axis=1)
    hw = jnp.dot(h0, w1[...], preferred_element_type=F32) + b1[...]
    for ci in range(8):
        e = (bd0[(ci >> 2) & 1:((ci >> 2) & 1) + 1, :]
             + bd1[(ci >> 1) & 1:((ci >> 1) & 1) + 1, :]
             + bd2[ci & 1:(ci & 1) + 1, :])
        t_out[ci] = dinv * jnp.maximum(hw + e, 0.0)
    self_out[...] = jnp.maximum(hw + root1[...], 0.0) * rdeg


def _mid_body(part, self1, aux, h1pre_out, bn_out):
    i = pl.program_id(0)
    p = part[...]
    dinv = aux[:, 0:1]
    h = dinv * (p[0] + p[1]) + self1[...]
    h1pre_out[...] = h
    sm = jnp.sum(h, axis=0, keepdims=True)
    sq = jnp.sum(h * h, axis=0, keepdims=True)

    @pl.when(i == 0)
    def _():
        bn_out[...] = jnp.zeros((8, HID), F32)

    bn_out[0:1, :] = bn_out[0:1, :] + sm
    bn_out[1:2, :] = bn_out[1:2, :] + sq


def _l2_body(h1pre, bn, aux, bd0, bd1, bd2, w2, b2, root2, g1, be1,
             t_out, self_out):
    mean = bn[0:1, :] * (1.0 / N)
    var = bn[1:2, :] * (1.0 / N) - mean * mean
    rstd = lax.rsqrt(var + EPS)
    h1 = jnp.maximum((h1pre[...] - mean) * rstd * g1[...] + be1[...], 0.0)
    dinv = aux[:, 0:1]
    rdeg = aux[:, 1:2]
    hw = jnp.dot(h1, w2[...], preferred_element_type=F32) + b2[...]
    for ci in range(8):
        e = (bd0[(ci >> 2) & 1:((ci >> 2) & 1) + 1, :]
             + bd1[(ci >> 1) & 1:((ci >> 1) & 1) + 1, :]
             + bd2[ci & 1:(ci & 1) + 1, :])
        t_out[ci] = dinv * jnp.maximum(hw + e, 0.0)
    self_out[...] = jnp.maximum(hw + root2[...], 0.0) * rdeg


def _pool_body(part, self2, aux, batr, pool_out, bn_out):
    i = pl.program_id(0)
    p = part[...]
    dinv = aux[:, 0:1]
    h = dinv * (p[0] + p[1]) + self2[...]
    sm = jnp.sum(h, axis=0, keepdims=True)
    sq = jnp.sum(h * h, axis=0, keepdims=True)

    @pl.when(i == 0)
    def _():
        bn_out[...] = jnp.zeros((8, HID), F32)
        pool_out[...] = jnp.zeros((G, HID), F32)

    bn_out[0:1, :] = bn_out[0:1, :] + sm
    bn_out[1:2, :] = bn_out[1:2, :] + sq
    b = batr[0]
    oh = (lax.broadcasted_iota(I32, (G, 1000), 0) == b).astype(F32)
    pool_out[...] = pool_out[...] + jnp.dot(oh, h, preferred_element_type=F32)


def _head_body(pool, bn, cntp, wp, bp, g2, be2, out):
    cnt = cntp[0, 0:G, 0:1] + cntp[1, 0:G, 0:1]
    mean = bn[0:1, :] * (1.0 / N)
    var = bn[1:2, :] * (1.0 / N) - mean * mean
    rstd = lax.rsqrt(var + EPS)
    hgr = pool[...] / jnp.maximum(cnt, 1.0)
    hg = (hgr - mean) * rstd * g2[...] + be2[...]
    hg = jnp.where(cnt > 0.0, hg, 0.0)
    out[...] = jnp.dot(hg, wp[...], preferred_element_type=F32) + bp[...]


def _full(shape):
    nd = len(shape)
    return pl.BlockSpec(shape, lambda i, _nd=nd: (0,) * _nd)


_DP_SPEC = pl.BlockSpec((NC, 1000, HID), lambda i: (0, i, 0))
_ROW_SPEC = pl.BlockSpec((1000, HID), lambda i: (i, 0))
_AUX_SPEC = pl.BlockSpec((1000, 8), lambda i: (i, 0))
_T_SPEC = pl.BlockSpec((8, 1000, HID), lambda i: (0, i, 0))
_ACC8_SPEC = pl.BlockSpec((8, HID), lambda i: (0, 0))


def kernel(params, x, edge_index, edge_attr, batch):
    p = params
    l1, l2 = p['layers'][0], p['layers'][1]

    xf = jnp.pad(x.astype(F32), ((0, 0), (0, 7)))
    row = edge_index[0].astype(I32)
    col = edge_index[1].astype(I32)
    ea = edge_attr.astype(I32)
    code = ea[:, 0] * 4 + ea[:, 1] * 2 + ea[:, 2]
    gidx = code * N + row
    padn = EPAD - E
    padn_e = EPAD_E - E
    gidx1d = jnp.pad(gidx, (0, padn_e))
    col1d = jnp.pad(col, (0, padn_e), constant_values=NODE_DUMMY)
    rowi2d = jnp.pad(row, (0, padn),
                     constant_values=NODE_DUMMY).reshape(EPAD // 128, 128)
    bat2d = jnp.pad(batch.astype(I32), (0, GPAD - N),
                    constant_values=G).reshape(GPAD // 128, 128)
    batr = batch.astype(I32).reshape(NB, 1, 1000)
    zrow = jnp.zeros((128, HID), F32)
    ones = jnp.ones((128, HID), F32)
    r1 = lambda a: a.reshape(1, -1)

    degp, cntp = _hist()(rowi2d, bat2d, zrow, ones)

    t1, self1, aux = pl.pallas_call(
        _enc_body,
        grid=(NB,),
        in_specs=[pl.BlockSpec((1000, 16), lambda i: (i, 0)), _DP_SPEC]
                 + [_full(t.shape) for t in p['atom']]
                 + [_full(t.shape) for t in l1['bond']]
                 + [_full((HID, HID)), _full((1, HID)), _full((1, HID))],
        out_specs=[_T_SPEC, _ROW_SPEC, _AUX_SPEC],
        out_shape=[jax.ShapeDtypeStruct((8, N, HID), F32),
                   jax.ShapeDtypeStruct((N, HID), F32),
                   jax.ShapeDtypeStruct((N, 8), F32)],
    )(xf, degp, *p['atom'], *l1['bond'], l1['W'], r1(l1['b']), l1['root'])

    part1 = _edge()(t1.reshape(8 * N, HID), gidx1d, col1d, zrow)

    h1pre, bn1 = pl.pallas_call(
        _mid_body,
        grid=(NB,),
        in_specs=[_PART_SPEC := pl.BlockSpec((NC, 1000, HID),
                                             lambda i: (0, i, 0)),
                  _ROW_SPEC, _AUX_SPEC],
        out_specs=[_ROW_SPEC, _ACC8_SPEC],
        out_shape=[jax.ShapeDtypeStruct((N, HID), F32),
                   jax.ShapeDtypeStruct((8, HID), F32)],
    )(part1, self1, aux)

    t2, self2 = pl.pallas_call(
        _l2_body,
        grid=(NB,),
        in_specs=[_ROW_SPEC, _ACC8_SPEC, _AUX_SPEC]
                 + [_full(t.shape) for t in l2['bond']]
                 + [_full((HID, HID)), _full((1, HID)), _full((1, HID)),
                    _full((1, HID)), _full((1, HID))],
        out_specs=[_T_SPEC, _ROW_SPEC],
        out_shape=[jax.ShapeDtypeStruct((8, N, HID), F32),
                   jax.ShapeDtypeStruct((N, HID), F32)],
    )(h1pre, bn1, aux, *l2['bond'], l2['W'], r1(l2['b']), l2['root'],
      r1(l1['bn_gamma']), r1(l1['bn_beta']))

    part2 = _edge()(t2.reshape(8 * N, HID), gidx1d, col1d, zrow)

    pool, bn2 = pl.pallas_call(
        _pool_body,
        grid=(NB,),
        in_specs=[_PART_SPEC, _ROW_SPEC, _AUX_SPEC,
                  pl.BlockSpec((1, 1, 1000), lambda i: (i, 0, 0))],
        out_specs=[pl.BlockSpec((G, HID), lambda i: (0, 0)), _ACC8_SPEC],
        out_shape=[jax.ShapeDtypeStruct((G, HID), F32),
                   jax.ShapeDtypeStruct((8, HID), F32)],
    )(part2, self2, aux, batr)

    out = pl.pallas_call(
        _head_body,
        grid=(1,),
        in_specs=[_full((G, HID)), _ACC8_SPEC, _full((NC, 512, HID)),
                  _full((HID, HID)), _full((1, HID)),
                  _full((1, HID)), _full((1, HID))],
        out_specs=pl.BlockSpec((G, HID), lambda i: (0, 0)),
        out_shape=jax.ShapeDtypeStruct((G, HID), F32),
    )(pool, bn2, cntp, p['Wp'], r1(p['bp']),
      r1(l2['bn_gamma']), r1(l2['bn_beta']))
    return out
